# baseline (speedup 1.0000x reference)
import functools

import jax
import jax.numpy as jnp
from jax.experimental import pallas as pl
from jax.experimental.pallas import tpu as pltpu

_N_SYS = 2048
_EPS = 1e-6


def _seg_kernel(sis_off_ref, pemd_off_ref,
                s_m2cx_ref, s_m2cy_ref, s_cc_ref, s_th_ref, s_cx_ref,
                s_cy_ref,
                p_m2cx_ref, p_m2cy_ref, p_cc_ref, p_b0_ref, p_b1_ref,
                p_cx_ref, p_cy_ref,
                g2_ref, gx_ref, gy_ref, out_ref, *, rr, cc, bsys, base):
    s = pl.program_id(0)
    g2 = g2_ref[...]
    gxp = gx_ref[...]
    gyp = gy_ref[...]

    def sis_body(i, carry):
        a, bx, by = carry
        u = g2 + s_cc_ref[i]
        u = u + s_m2cx_ref[i] * gxp
        u = u + s_m2cy_ref[i] * gyp
        coef = s_th_ref[i] * jax.lax.rsqrt(u)
        return a + coef, bx + s_cx_ref[i] * coef, by + s_cy_ref[i] * coef

    def pemd_body(i, carry):
        a, bx, by = carry
        u = g2 + p_cc_ref[i]
        u = u + p_m2cx_ref[i] * gxp
        u = u + p_m2cy_ref[i] * gyp
        coef = jnp.exp(p_b0_ref[i] + p_b1_ref[i] * jnp.log(u))
        coef = coef * jax.lax.rsqrt(u)
        return a + coef, bx + p_cx_ref[i] * coef, by + p_cy_ref[i] * coef

    zero = jnp.zeros((rr, cc), jnp.float32)
    for j in range(bsys):
        sysid = base + s * bsys + j
        carry = jax.lax.fori_loop(sis_off_ref[sysid], sis_off_ref[sysid + 1],
                                  sis_body, (zero, zero, zero))
        a, bx, by = jax.lax.fori_loop(pemd_off_ref[sysid],
                                      pemd_off_ref[sysid + 1], pemd_body,
                                      carry)
        na = 1.0 - a
        out_ref[j, 0] = gxp * na + bx
        out_ref[j, 1] = gyp * na + by


def _offsets(idx):
    counts = jnp.bincount(idx, length=_N_SYS)
    return jnp.concatenate(
        [jnp.zeros((1,), jnp.int32),
         jnp.cumsum(counts).astype(jnp.int32)])


@jax.jit
def kernel(lens_grid, sis_params, pemd_params, sis_idx, pemd_idx):
    hh, ww, _ = lens_grid.shape
    rr = hh * ww // 128
    gx = lens_grid[:, :, 0].reshape(rr, 128)
    gy = lens_grid[:, :, 1].reshape(rr, 128)
    g2 = gx * gx + gy * gy + _EPS

    so = jnp.argsort(sis_idx)
    sp = sis_params[so]
    s_th, s_cx, s_cy = sp[:, 0], sp[:, 1], sp[:, 2]
    s_m2cx = -2.0 * s_cx
    s_m2cy = -2.0 * s_cy
    s_cc = s_cx * s_cx + s_cy * s_cy
    sis_off = _offsets(sis_idx)

    po = jnp.argsort(pemd_idx)
    pp = pemd_params[po]
    th, gam, p_cx, p_cy = pp[:, 0], pp[:, 1], pp[:, 2], pp[:, 3]
    p_b0 = (gam - 1.0) * jnp.log(th)
    p_b1 = 0.5 * (2.0 - gam)
    p_m2cx = -2.0 * p_cx
    p_m2cy = -2.0 * p_cy
    p_cc = p_cx * p_cx + p_cy * p_cy
    pemd_off = _offsets(pemd_idx)

    bsys = 16
    nchunks = 2
    csys = _N_SYS // nchunks
    chunks = []
    for c in range(nchunks):
        out_c = pl.pallas_call(
            functools.partial(_seg_kernel, rr=rr, cc=128, bsys=bsys,
                              base=c * csys),
            grid=(csys // bsys,),
            in_specs=[pl.BlockSpec(memory_space=pltpu.SMEM)] * 15 + [
                pl.BlockSpec((rr, 128), lambda s: (0, 0)),
                pl.BlockSpec((rr, 128), lambda s: (0, 0)),
                pl.BlockSpec((rr, 128), lambda s: (0, 0)),
            ],
            out_specs=pl.BlockSpec((bsys, 2, rr, 128),
                                   lambda s: (s, 0, 0, 0)),
            out_shape=jax.ShapeDtypeStruct((csys, 2, rr, 128), jnp.float32),
        )(sis_off, pemd_off,
          s_m2cx, s_m2cy, s_cc, s_th, s_cx, s_cy,
          p_m2cx, p_m2cy, p_cc, p_b0, p_b1, p_cx, p_cy,
          g2, gx, gy)
        chunks.append(out_c.reshape(csys, 2, hh, ww).transpose(0, 2, 3, 1))
    return jnp.concatenate(chunks, axis=0)


# speedup vs baseline: 1.1310x; 1.1310x over previous
import functools

import jax
import jax.numpy as jnp
from jax.experimental import pallas as pl
from jax.experimental.pallas import tpu as pltpu

_N_SYS = 2048
_EPS = 1e-6


def _seg_kernel(sis_off_ref, pemd_off_ref,
                s_m2cx_ref, s_m2cy_ref, s_cc_ref, s_th_ref, s_cx_ref,
                s_cy_ref,
                p_m2cx_ref, p_m2cy_ref, p_cc_ref, p_b0_ref, p_b1_ref,
                p_cx_ref, p_cy_ref,
                g2_ref, gx_ref, gy_ref, out_ref, *, rr, cc, bsys, base):
    s = pl.program_id(0)
    g2 = g2_ref[...]
    gxp = gx_ref[...]
    gyp = gy_ref[...]

    def sis_body(i, carry):
        a, bx, by = carry
        u = g2 + s_cc_ref[i]
        u = u + s_m2cx_ref[i] * gxp
        u = u + s_m2cy_ref[i] * gyp
        coef = s_th_ref[i] * jax.lax.rsqrt(u)
        return a + coef, bx + s_cx_ref[i] * coef, by + s_cy_ref[i] * coef

    def pemd_body(i, carry):
        a, bx, by = carry
        u = g2 + p_cc_ref[i]
        u = u + p_m2cx_ref[i] * gxp
        u = u + p_m2cy_ref[i] * gyp
        coef = jnp.exp(p_b0_ref[i] + p_b1_ref[i] * jnp.log(u))
        coef = coef * jax.lax.rsqrt(u)
        return a + coef, bx + p_cx_ref[i] * coef, by + p_cy_ref[i] * coef

    zero = jnp.zeros((rr, cc), jnp.float32)
    for j in range(bsys):
        sysid = base + s * bsys + j
        carry = jax.lax.fori_loop(sis_off_ref[sysid], sis_off_ref[sysid + 1],
                                  sis_body, (zero, zero, zero))
        a, bx, by = jax.lax.fori_loop(pemd_off_ref[sysid],
                                      pemd_off_ref[sysid + 1], pemd_body,
                                      carry)
        na = 1.0 - a
        out_ref[j, 0] = gxp * na + bx
        out_ref[j, 1] = gyp * na + by


def _offsets(idx):
    counts = jnp.bincount(idx, length=_N_SYS)
    return jnp.concatenate(
        [jnp.zeros((1,), jnp.int32),
         jnp.cumsum(counts).astype(jnp.int32)])


@jax.jit
def kernel(lens_grid, sis_params, pemd_params, sis_idx, pemd_idx):
    hh, ww, _ = lens_grid.shape
    rr = hh * ww // 128
    gx = lens_grid[:, :, 0].reshape(rr, 128)
    gy = lens_grid[:, :, 1].reshape(rr, 128)
    g2 = gx * gx + gy * gy + _EPS

    so = jnp.argsort(sis_idx)
    sp = sis_params[so]
    s_th, s_cx, s_cy = sp[:, 0], sp[:, 1], sp[:, 2]
    s_m2cx = -2.0 * s_cx
    s_m2cy = -2.0 * s_cy
    s_cc = s_cx * s_cx + s_cy * s_cy
    sis_off = _offsets(sis_idx)

    po = jnp.argsort(pemd_idx)
    pp = pemd_params[po]
    th, gam, p_cx, p_cy = pp[:, 0], pp[:, 1], pp[:, 2], pp[:, 3]
    p_b0 = (gam - 1.0) * jnp.log(th)
    p_b1 = 0.5 * (2.0 - gam)
    p_m2cx = -2.0 * p_cx
    p_m2cy = -2.0 * p_cy
    p_cc = p_cx * p_cx + p_cy * p_cy
    pemd_off = _offsets(pemd_idx)

    bsys = 16
    out = pl.pallas_call(
        functools.partial(_seg_kernel, rr=rr, cc=128, bsys=bsys, base=0),
        grid=(_N_SYS // bsys,),
        in_specs=[pl.BlockSpec(memory_space=pltpu.SMEM)] * 15 + [
            pl.BlockSpec((rr, 128), lambda s: (0, 0)),
            pl.BlockSpec((rr, 128), lambda s: (0, 0)),
            pl.BlockSpec((rr, 128), lambda s: (0, 0)),
        ],
        out_specs=pl.BlockSpec((bsys, 2, rr, 128), lambda s: (s, 0, 0, 0)),
        out_shape=jax.ShapeDtypeStruct((_N_SYS, 2, rr, 128), jnp.float32),
    )(sis_off, pemd_off,
      s_m2cx, s_m2cy, s_cc, s_th, s_cx, s_cy,
      p_m2cx, p_m2cy, p_cc, p_b0, p_b1, p_cx, p_cy,
      g2, gx, gy)
    return out.reshape(_N_SYS, 2, hh, ww).transpose(0, 2, 3, 1)
